# baseline (speedup 1.0000x reference)
import jax
import jax.numpy as jnp
from jax import lax
from jax.experimental import pallas as pl
from jax.experimental.pallas import tpu as pltpu
from jax.experimental.pallas import tpu_sc as plsc

DIM = 16
BATCH = 16384
FIELDS = 26
B = BATCH * FIELDS
NC, NS = 2, 16
NW = NC * NS
BPW = B // NW
CHUNK = 128
NV = BPW // CHUNK


def _emb_body(x_hbm, f_hbm, l_hbm, out_hbm, idx_v, buf_f, buf_l, sem_f, sem_l):
    wid = lax.axis_index("s") * NC + lax.axis_index("c")
    base = wid * BPW
    pltpu.sync_copy(x_hbm.at[wid], idx_v)

    def step(j, carry):
        row0 = base + j * CHUNK
        pltpu.async_copy(f_hbm.at[idx_v.at[j]], buf_f, sem_f).wait()
        pltpu.async_copy(l_hbm.at[idx_v.at[j]], buf_l, sem_l).wait()
        pltpu.sync_copy(buf_f, out_hbm.at[pl.ds(row0, CHUNK), pl.ds(0, DIM)])
        pltpu.sync_copy(buf_l, out_hbm.at[pl.ds(row0, CHUNK), pl.ds(DIM, DIM)])
        return carry

    lax.fori_loop(0, NV, step, 0)


@jax.jit
def _run(x_flat, frozen_table, learn_table):
    mesh = plsc.VectorSubcoreMesh(core_axis_name="c", subcore_axis_name="s")
    k = pl.kernel(
        _emb_body,
        mesh=mesh,
        out_type=jax.ShapeDtypeStruct((B, 2 * DIM), jnp.float32),
        scratch_types=[
            pltpu.VMEM((NV, CHUNK), jnp.int32),
            pltpu.VMEM((CHUNK, DIM), jnp.float32),
            pltpu.VMEM((CHUNK, DIM), jnp.float32),
            pltpu.SemaphoreType.DMA,
            pltpu.SemaphoreType.DMA,
        ],
        compiler_params=pltpu.CompilerParams(use_tc_tiling_on_sc=False),
    )
    return k(x_flat, frozen_table, learn_table)


def kernel(x, frozen_table, learn_table):
    x_flat = x.astype(jnp.int32).reshape(NW, NV, CHUNK)
    out = _run(x_flat, frozen_table, learn_table)
    return out.reshape(BATCH, FIELDS, 2 * DIM)


# speedup vs baseline: 1.1409x; 1.1409x over previous
import jax
import jax.numpy as jnp
from jax import lax
from jax.experimental import pallas as pl
from jax.experimental.pallas import tpu as pltpu
from jax.experimental.pallas import tpu_sc as plsc

DIM = 16
BATCH = 16384
FIELDS = 26
B = BATCH * FIELDS
NC, NS = 2, 16
NW = NC * NS
BPW = B // NW
CHUNK = 128
NV = BPW // CHUNK
SUP = 4
ROWS = SUP * CHUNK
NSUP = NV // SUP


def _emb_body(x_hbm, f_hbm, l_hbm, out_hbm,
              idx_v, bf0, bf1, bl0, bl1, gsem0, gsem1, ssem0, ssem1):
    wid = lax.axis_index("s") * NC + lax.axis_index("c")
    base = wid * BPW
    pltpu.sync_copy(x_hbm.at[wid], idx_v)

    bufs_f = (bf0, bf1)
    bufs_l = (bl0, bl1)
    gsems = (gsem0, gsem1)
    ssems = (ssem0, ssem1)

    def fire_gathers(s, slot):
        for v in range(SUP):
            pltpu.async_copy(f_hbm.at[idx_v.at[s * SUP + v]],
                             bufs_f[slot].at[pl.ds(v * CHUNK, CHUNK)],
                             gsems[slot])
            pltpu.async_copy(l_hbm.at[idx_v.at[s * SUP + v]],
                             bufs_l[slot].at[pl.ds(v * CHUNK, CHUNK)],
                             gsems[slot])

    def wait_gathers(s, slot):
        for v in range(SUP):
            pltpu.make_async_copy(f_hbm.at[idx_v.at[s * SUP + v]],
                                  bufs_f[slot].at[pl.ds(v * CHUNK, CHUNK)],
                                  gsems[slot]).wait()
            pltpu.make_async_copy(l_hbm.at[idx_v.at[s * SUP + v]],
                                  bufs_l[slot].at[pl.ds(v * CHUNK, CHUNK)],
                                  gsems[slot]).wait()

    def fire_scatters(s, slot):
        row0 = base + s * ROWS
        pltpu.async_copy(bufs_f[slot],
                         out_hbm.at[pl.ds(row0, ROWS), pl.ds(0, DIM)],
                         ssems[slot])
        pltpu.async_copy(bufs_l[slot],
                         out_hbm.at[pl.ds(row0, ROWS), pl.ds(DIM, DIM)],
                         ssems[slot])

    def wait_scatters(s, slot):
        row0 = base + s * ROWS
        pltpu.make_async_copy(bufs_f[slot],
                              out_hbm.at[pl.ds(row0, ROWS), pl.ds(0, DIM)],
                              ssems[slot]).wait()
        pltpu.make_async_copy(bufs_l[slot],
                              out_hbm.at[pl.ds(row0, ROWS), pl.ds(DIM, DIM)],
                              ssems[slot]).wait()

    def body(i, carry):
        for b in range(2):
            s = 2 * i + b
            @pl.when(i >= 1)
            def _():
                wait_scatters(s - 2, b)
            fire_gathers(s, b)
            if b == 1:
                wait_gathers(s - 1, 0)
                fire_scatters(s - 1, 0)
            else:
                @pl.when(i >= 1)
                def _():
                    wait_gathers(s - 1, 1)
                    fire_scatters(s - 1, 1)
        return carry

    lax.fori_loop(0, NSUP // 2, body, 0)
    wait_gathers(NSUP - 1, 1)
    fire_scatters(NSUP - 1, 1)
    wait_scatters(NSUP - 2, 0)
    wait_scatters(NSUP - 1, 1)


@jax.jit
def _run(x_flat, frozen_table, learn_table):
    mesh = plsc.VectorSubcoreMesh(core_axis_name="c", subcore_axis_name="s")
    k = pl.kernel(
        _emb_body,
        mesh=mesh,
        out_type=jax.ShapeDtypeStruct((B, 2 * DIM), jnp.float32),
        scratch_types=[
            pltpu.VMEM((NV, CHUNK), jnp.int32),
            pltpu.VMEM((ROWS, DIM), jnp.float32),
            pltpu.VMEM((ROWS, DIM), jnp.float32),
            pltpu.VMEM((ROWS, DIM), jnp.float32),
            pltpu.VMEM((ROWS, DIM), jnp.float32),
            pltpu.SemaphoreType.DMA,
            pltpu.SemaphoreType.DMA,
            pltpu.SemaphoreType.DMA,
            pltpu.SemaphoreType.DMA,
        ],
        compiler_params=pltpu.CompilerParams(use_tc_tiling_on_sc=False),
    )
    return k(x_flat, frozen_table, learn_table)


def kernel(x, frozen_table, learn_table):
    x_flat = x.astype(jnp.int32).reshape(NW, NV, CHUNK)
    out = _run(x_flat, frozen_table, learn_table)
    return out.reshape(BATCH, FIELDS, 2 * DIM)


# speedup vs baseline: 1.2148x; 1.0648x over previous
import jax
import jax.numpy as jnp
from jax import lax
from jax.experimental import pallas as pl
from jax.experimental.pallas import tpu as pltpu
from jax.experimental.pallas import tpu_sc as plsc

DIM = 16
BATCH = 16384
FIELDS = 26
NC, NS = 2, 16
NW = NC * NS
LANE = 128
BBLK = 4
NBLK = FIELDS * BBLK


def _emb_body(x_hbm, f_hbm, l_hbm, out_hbm,
              idx_v, bf0, bf1, bl0, bl1, tiles0, tiles1,
              gsem0, gsem1, wsem0, wsem1):
    wid = lax.axis_index("s") * NC + lax.axis_index("c")
    pltpu.sync_copy(x_hbm.at[wid], idx_v)

    bufs_f = (bf0, bf1)
    bufs_l = (bl0, bl1)
    tiles = (tiles0, tiles1)
    gsems = (gsem0, gsem1)
    wsems = (wsem0, wsem1)

    rows16 = jnp.arange(16, dtype=jnp.int32)
    row_vecs = [rows16 + 16 * g for g in range(8)]
    col_vecs = [jnp.full((16,), s, jnp.int32) for s in range(16)]

    def fire_gathers(p, slot):
        f = p // BBLK
        j = p % BBLK
        pltpu.async_copy(f_hbm.at[idx_v.at[f, j]], bufs_f[slot], gsems[slot])
        pltpu.async_copy(l_hbm.at[idx_v.at[f, j]], bufs_l[slot], gsems[slot])

    def wait_gathers(p, slot):
        f = p // BBLK
        j = p % BBLK
        pltpu.make_async_copy(f_hbm.at[idx_v.at[f, j]], bufs_f[slot],
                              gsems[slot]).wait()
        pltpu.make_async_copy(l_hbm.at[idx_v.at[f, j]], bufs_l[slot],
                              gsems[slot]).wait()

    def transpose_block(slot):
        t = tiles[slot]
        for half in range(2):
            for s in range(8):
                c = col_vecs[half * 8 + s]
                for g in range(8):
                    v = plsc.load_gather(bufs_f[slot], [row_vecs[g], c])
                    t[half, s, pl.ds(g * 16, 16)] = v
                    v = plsc.load_gather(bufs_l[slot], [row_vecs[g], c])
                    t[2 + half, s, pl.ds(g * 16, 16)] = v

    def fire_writes(p, slot):
        f = p // BBLK
        btg = wid * BBLK + (p % BBLK)
        for dt in range(4):
            pltpu.async_copy(tiles[slot].at[dt], out_hbm.at[f, dt, btg],
                             wsems[slot])

    def wait_writes(p, slot):
        f = p // BBLK
        btg = wid * BBLK + (p % BBLK)
        for dt in range(4):
            pltpu.make_async_copy(tiles[slot].at[dt], out_hbm.at[f, dt, btg],
                                  wsems[slot]).wait()

    fire_gathers(0, 0)

    def body(i, carry):
        for b in range(2):
            p = 2 * i + b
            @pl.when(p + 1 < NBLK)
            def _():
                fire_gathers(p + 1, 1 - b)
            wait_gathers(p, b)
            @pl.when(i >= 1)
            def _():
                wait_writes(p - 2, b)
            transpose_block(b)
            fire_writes(p, b)
        return carry

    lax.fori_loop(0, NBLK // 2, body, 0)
    wait_writes(NBLK - 2, 0)
    wait_writes(NBLK - 1, 1)


@jax.jit
def _run(x_w, frozen_table, learn_table):
    mesh = plsc.VectorSubcoreMesh(core_axis_name="c", subcore_axis_name="s")
    k = pl.kernel(
        _emb_body,
        mesh=mesh,
        out_type=jax.ShapeDtypeStruct((FIELDS, 4, BATCH // LANE, 8, LANE),
                                      jnp.float32),
        scratch_types=[
            pltpu.VMEM((FIELDS, BBLK, LANE), jnp.int32),
            pltpu.VMEM((LANE, DIM), jnp.float32),
            pltpu.VMEM((LANE, DIM), jnp.float32),
            pltpu.VMEM((LANE, DIM), jnp.float32),
            pltpu.VMEM((LANE, DIM), jnp.float32),
            pltpu.VMEM((4, 8, LANE), jnp.float32),
            pltpu.VMEM((4, 8, LANE), jnp.float32),
            pltpu.SemaphoreType.DMA,
            pltpu.SemaphoreType.DMA,
            pltpu.SemaphoreType.DMA,
            pltpu.SemaphoreType.DMA,
        ],
        compiler_params=pltpu.CompilerParams(use_tc_tiling_on_sc=False,
                                             needs_layout_passes=False),
    )
    return k(x_w, frozen_table, learn_table)


def kernel(x, frozen_table, learn_table):
    x_w = (x.astype(jnp.int32).T
           .reshape(FIELDS, NW, BBLK, LANE).transpose(1, 0, 2, 3))
    out = _run(x_w, frozen_table, learn_table)
    return out.transpose(2, 4, 0, 1, 3).reshape(BATCH, FIELDS, 2 * DIM)


# speedup vs baseline: 1.3568x; 1.1169x over previous
import jax
import jax.numpy as jnp
from jax import lax
from jax.experimental import pallas as pl
from jax.experimental.pallas import tpu as pltpu
from jax.experimental.pallas import tpu_sc as plsc

DIM = 16
BATCH = 16384
FIELDS = 26
NC, NS = 2, 16
NW = NC * NS
LANE = 128
BBLK = 4
NBLK = FIELDS * BBLK


def _emb_body(x_hbm, f_hbm, l_hbm, out_hbm,
              idx_v, bf0, bf1, bl0, bl1, tiles0, tiles1,
              gsem0, gsem1, wsem0, wsem1):
    wid = lax.axis_index("s") * NC + lax.axis_index("c")
    pltpu.sync_copy(x_hbm.at[wid], idx_v)

    bufs_f = (bf0, bf1)
    bufs_l = (bl0, bl1)
    tiles = (tiles0, tiles1)
    gsems = (gsem0, gsem1)
    wsems = (wsem0, wsem1)

    iota16 = jnp.arange(16, dtype=jnp.int32)
    iota16h = iota16 + 16

    def fire_gathers(p, slot):
        f = p // BBLK
        j = p % BBLK
        pltpu.async_copy(f_hbm.at[idx_v.at[f, j]], bufs_f[slot], gsems[slot])
        pltpu.async_copy(l_hbm.at[idx_v.at[f, j]], bufs_l[slot], gsems[slot])

    def wait_gathers(p, slot):
        f = p // BBLK
        j = p % BBLK
        pltpu.make_async_copy(f_hbm.at[idx_v.at[f, j]], bufs_f[slot],
                              gsems[slot]).wait()
        pltpu.make_async_copy(l_hbm.at[idx_v.at[f, j]], bufs_l[slot],
                              gsems[slot]).wait()

    def transpose_block(slot):
        t = tiles[slot]
        for l in range(LANE):
            lv = jnp.full((16,), l, jnp.int32)
            plsc.store_scatter(t, [iota16, lv], bufs_f[slot][l, :])
            plsc.store_scatter(t, [iota16h, lv], bufs_l[slot][l, :])

    def fire_writes(p, slot):
        f = p // BBLK
        btg = wid * BBLK + (p % BBLK)
        for dt in range(4):
            pltpu.async_copy(tiles[slot].at[pl.ds(dt * 8, 8), pl.ds(0, LANE)],
                             out_hbm.at[f, dt, btg], wsems[slot])

    def wait_writes(p, slot):
        f = p // BBLK
        btg = wid * BBLK + (p % BBLK)
        for dt in range(4):
            pltpu.make_async_copy(
                tiles[slot].at[pl.ds(dt * 8, 8), pl.ds(0, LANE)],
                out_hbm.at[f, dt, btg], wsems[slot]).wait()

    fire_gathers(0, 0)

    def body(i, carry):
        for b in range(2):
            p = 2 * i + b
            @pl.when(p + 1 < NBLK)
            def _():
                fire_gathers(p + 1, 1 - b)
            wait_gathers(p, b)
            @pl.when(i >= 1)
            def _():
                wait_writes(p - 2, b)
            transpose_block(b)
            fire_writes(p, b)
        return carry

    lax.fori_loop(0, NBLK // 2, body, 0)
    wait_writes(NBLK - 2, 0)
    wait_writes(NBLK - 1, 1)


@jax.jit
def _run(x_w, frozen_table, learn_table):
    mesh = plsc.VectorSubcoreMesh(core_axis_name="c", subcore_axis_name="s")
    k = pl.kernel(
        _emb_body,
        mesh=mesh,
        out_type=jax.ShapeDtypeStruct((FIELDS, 4, BATCH // LANE, 8, LANE),
                                      jnp.float32),
        scratch_types=[
            pltpu.VMEM((FIELDS, BBLK, LANE), jnp.int32),
            pltpu.VMEM((LANE, DIM), jnp.float32),
            pltpu.VMEM((LANE, DIM), jnp.float32),
            pltpu.VMEM((LANE, DIM), jnp.float32),
            pltpu.VMEM((LANE, DIM), jnp.float32),
            pltpu.VMEM((32, LANE + 1), jnp.float32),
            pltpu.VMEM((32, LANE + 1), jnp.float32),
            pltpu.SemaphoreType.DMA,
            pltpu.SemaphoreType.DMA,
            pltpu.SemaphoreType.DMA,
            pltpu.SemaphoreType.DMA,
        ],
        compiler_params=pltpu.CompilerParams(use_tc_tiling_on_sc=False,
                                             needs_layout_passes=False),
    )
    return k(x_w, frozen_table, learn_table)


def kernel(x, frozen_table, learn_table):
    x_w = (x.astype(jnp.int32).T
           .reshape(FIELDS, NW, BBLK, LANE).transpose(1, 0, 2, 3))
    out = _run(x_w, frozen_table, learn_table)
    return out.transpose(2, 4, 0, 1, 3).reshape(BATCH, FIELDS, 2 * DIM)


# speedup vs baseline: 1.3618x; 1.0036x over previous
import functools

import jax
import jax.numpy as jnp
from jax import lax
from jax.experimental import pallas as pl
from jax.experimental.pallas import tpu as pltpu
from jax.experimental.pallas import tpu_sc as plsc

DIM = 16
BATCH = 16384
FIELDS = 26
NC, NS = 2, 16
NW = NC * NS
LANE = 128
BBLK = 4
NBLK = FIELDS * BBLK


def _emb_body(x_hbm, f_hbm, l_hbm, out_hbm,
              idx_v, bf0, bf1, bl0, bl1, tiles0, tiles1,
              gsem0, gsem1, wsem0, wsem1):
    wid = lax.axis_index("s") * NC + lax.axis_index("c")
    pltpu.sync_copy(x_hbm.at[wid], idx_v)

    bufs_f = (bf0, bf1)
    bufs_l = (bl0, bl1)
    tiles = (tiles0, tiles1)
    gsems = (gsem0, gsem1)
    wsems = (wsem0, wsem1)

    iota16 = jnp.arange(16, dtype=jnp.int32)
    iota16h = iota16 + 16

    def fire_gathers(p, slot):
        f = p // BBLK
        j = p % BBLK
        pltpu.async_copy(f_hbm.at[idx_v.at[f, j]], bufs_f[slot], gsems[slot])
        pltpu.async_copy(l_hbm.at[idx_v.at[f, j]], bufs_l[slot], gsems[slot])

    def wait_gathers(p, slot):
        f = p // BBLK
        j = p % BBLK
        pltpu.make_async_copy(f_hbm.at[idx_v.at[f, j]], bufs_f[slot],
                              gsems[slot]).wait()
        pltpu.make_async_copy(l_hbm.at[idx_v.at[f, j]], bufs_l[slot],
                              gsems[slot]).wait()

    def transpose_block(slot):
        t = tiles[slot]
        for l in range(LANE):
            lv = jnp.full((16,), l, jnp.int32)
            plsc.store_scatter(t, [iota16, lv], bufs_f[slot][l, :])
            plsc.store_scatter(t, [iota16h, lv], bufs_l[slot][l, :])

    def fire_writes(p, slot):
        f = p // BBLK
        btg = wid * BBLK + (p % BBLK)
        for dt in range(4):
            pltpu.async_copy(tiles[slot].at[pl.ds(dt * 8, 8), pl.ds(0, LANE)],
                             out_hbm.at[f, dt, btg], wsems[slot])

    def wait_writes(p, slot):
        f = p // BBLK
        btg = wid * BBLK + (p % BBLK)
        for dt in range(4):
            pltpu.make_async_copy(
                tiles[slot].at[pl.ds(dt * 8, 8), pl.ds(0, LANE)],
                out_hbm.at[f, dt, btg], wsems[slot]).wait()

    fire_gathers(0, 0)

    def body(i, carry):
        for b in range(2):
            p = 2 * i + b
            @pl.when(p + 1 < NBLK)
            def _():
                fire_gathers(p + 1, 1 - b)
            wait_gathers(p, b)
            @pl.when(i >= 1)
            def _():
                wait_writes(p - 2, b)
            transpose_block(b)
            fire_writes(p, b)
        return carry

    lax.fori_loop(0, NBLK // 2, body, 0)
    wait_writes(NBLK - 2, 0)
    wait_writes(NBLK - 1, 1)


@jax.jit
def _run(x_w, frozen_table, learn_table):
    mesh = plsc.VectorSubcoreMesh(core_axis_name="c", subcore_axis_name="s")
    k = pl.kernel(
        _emb_body,
        mesh=mesh,
        out_type=jax.ShapeDtypeStruct((FIELDS, 4, BATCH // LANE, 8, LANE),
                                      jnp.float32),
        scratch_types=[
            pltpu.VMEM((FIELDS, BBLK, LANE), jnp.int32),
            pltpu.VMEM((LANE, DIM), jnp.float32),
            pltpu.VMEM((LANE, DIM), jnp.float32),
            pltpu.VMEM((LANE, DIM), jnp.float32),
            pltpu.VMEM((LANE, DIM), jnp.float32),
            pltpu.VMEM((32, LANE + 1), jnp.float32),
            pltpu.VMEM((32, LANE + 1), jnp.float32),
            pltpu.SemaphoreType.DMA,
            pltpu.SemaphoreType.DMA,
            pltpu.SemaphoreType.DMA,
            pltpu.SemaphoreType.DMA,
        ],
        compiler_params=pltpu.CompilerParams(use_tc_tiling_on_sc=False,
                                             needs_layout_passes=False),
    )
    return k(x_w, frozen_table, learn_table)


def kernel(x, frozen_table, learn_table):
    x_w = (x.astype(jnp.int32).T
           .reshape(FIELDS, NW, BBLK, LANE).transpose(1, 0, 2, 3))
    out = _run(x_w, frozen_table, learn_table)
    return out.transpose(2, 4, 0, 1, 3).reshape(BATCH, FIELDS, 2 * DIM)
